# baseline (speedup 1.0000x reference)
import functools

import jax
import jax.numpy as jnp
from jax.experimental import pallas as pl
from jax.experimental.pallas import tpu as pltpu


def _fused_kernel(x_ref, L_ref, wc_ref, bc_ref, wfc_ref, bfc_ref, out_ref,
                  *, K, G):
    Lb = L_ref[...].astype(jnp.bfloat16)
    dot = functools.partial(jnp.dot, preferred_element_type=jnp.float32)

    t_m2 = x_ref[...]
    tb = [t_m2.astype(jnp.bfloat16)]
    t_m1 = dot(tb[0], Lb)
    tb.append(t_m1.astype(jnp.bfloat16))
    for k in range(2, K):
        t = 2.0 * dot(tb[-1], Lb) - t_m2
        t_m2, t_m1 = t_m1, t
        tb.append(t.astype(jnp.bfloat16))

    wfcb = wfc_ref[...].astype(jnp.bfloat16)
    logits = bfc_ref[...]
    for g in range(G):
        h_g = wc_ref[0, g] * tb[0]
        for k in range(1, K):
            h_g = h_g + wc_ref[k, g] * tb[k]
        h_g = jnp.maximum(h_g + bc_ref[g], 0.0)
        logits = logits + dot(h_g.astype(jnp.bfloat16), wfcb[g])

    m = jnp.max(logits, axis=1, keepdims=True)
    s = logits - m
    out_ref[...] = s - jnp.log(jnp.sum(jnp.exp(s), axis=1, keepdims=True))


def kernel(x, L, W_cheb, b_cheb, W_fc, b_fc):
    B, N, F_IN = x.shape
    K, _, G = W_cheb.shape
    D = W_fc.shape[1]
    xt = x.reshape(B, N)
    wc = W_cheb.reshape(K, G)
    wfc = W_fc.reshape(N, G, D).transpose(1, 0, 2)

    fn = pl.pallas_call(
        functools.partial(_fused_kernel, K=K, G=G),
        out_shape=jax.ShapeDtypeStruct((B, D), jnp.float32),
        in_specs=[
            pl.BlockSpec(memory_space=pltpu.VMEM),
            pl.BlockSpec(memory_space=pltpu.VMEM),
            pl.BlockSpec(memory_space=pltpu.SMEM),
            pl.BlockSpec(memory_space=pltpu.SMEM),
            pl.BlockSpec(memory_space=pltpu.VMEM),
            pl.BlockSpec(memory_space=pltpu.VMEM),
        ],
        out_specs=pl.BlockSpec(memory_space=pltpu.VMEM),
    )
    return fn(xt, L, wc, b_cheb, wfc, b_fc.reshape(1, D))


# speedup vs baseline: 1.0249x; 1.0249x over previous
import functools

import jax
import jax.numpy as jnp
from jax.experimental import pallas as pl
from jax.experimental.pallas import tpu as pltpu


def _fused_kernel(x_ref, L_ref, wc_ref, bc_ref, wfc_ref, bfc_ref, out_ref,
                  *, K, G):
    Lb2 = (L_ref[...] * 2.0).astype(jnp.bfloat16)
    dot = functools.partial(jnp.dot, preferred_element_type=jnp.float32)

    CHUNK = 4
    t_m2 = x_ref[...]
    t_m1 = 0.5 * dot(t_m2.astype(jnp.bfloat16), Lb2)
    acc = [None] * G
    pend = [(0, t_m2), (1, t_m1)]

    def flush(pend, acc):
        for g in range(G):
            s = wc_ref[pend[0][0], g] * pend[0][1]
            for k, t in pend[1:]:
                s = s + wc_ref[k, g] * t
            acc[g] = s if acc[g] is None else acc[g] + s

    for k in range(2, K):
        t = dot(t_m1.astype(jnp.bfloat16), Lb2) - t_m2
        t_m2, t_m1 = t_m1, t
        pend.append((k, t))
        if len(pend) == CHUNK:
            flush(pend, acc)
            pend = []
    if pend:
        flush(pend, acc)

    wfcb = wfc_ref[...].astype(jnp.bfloat16)
    logits = bfc_ref[...]
    for g in range(G):
        h_g = jnp.maximum(acc[g] + bc_ref[g], 0.0)
        logits = logits + dot(h_g.astype(jnp.bfloat16), wfcb[g])

    m = jnp.max(logits, axis=1, keepdims=True)
    s = logits - m
    out_ref[...] = s - jnp.log(jnp.sum(jnp.exp(s), axis=1, keepdims=True))


def kernel(x, L, W_cheb, b_cheb, W_fc, b_fc):
    B, N, F_IN = x.shape
    K, _, G = W_cheb.shape
    D = W_fc.shape[1]
    xt = x.reshape(B, N)
    wc = W_cheb.reshape(K, G)
    wfc = W_fc.reshape(N, G, D).transpose(1, 0, 2)

    fn = pl.pallas_call(
        functools.partial(_fused_kernel, K=K, G=G),
        out_shape=jax.ShapeDtypeStruct((B, D), jnp.float32),
        in_specs=[
            pl.BlockSpec(memory_space=pltpu.VMEM),
            pl.BlockSpec(memory_space=pltpu.VMEM),
            pl.BlockSpec(memory_space=pltpu.SMEM),
            pl.BlockSpec(memory_space=pltpu.SMEM),
            pl.BlockSpec(memory_space=pltpu.VMEM),
            pl.BlockSpec(memory_space=pltpu.VMEM),
        ],
        out_specs=pl.BlockSpec(memory_space=pltpu.VMEM),
    )
    return fn(xt, L, wc, b_cheb, wfc, b_fc.reshape(1, D))


# speedup vs baseline: 1.1500x; 1.1221x over previous
import functools

import jax
import jax.numpy as jnp
from jax.experimental import pallas as pl
from jax.experimental.pallas import tpu as pltpu


def _fused_kernel(x_ref, L_ref, wc_ref, bc_ref, wfc_ref, bfc_ref, out_ref,
                  *, K, G):
    Lb2 = (L_ref[...] * 2.0).astype(jnp.bfloat16)
    dot = functools.partial(jnp.dot, preferred_element_type=jnp.float32)

    CHUNK = 4
    t_m2 = x_ref[...]
    tb = x_ref[...].astype(jnp.bfloat16)
    t_m1 = 0.5 * dot(tb, Lb2)
    tb1 = t_m1.astype(jnp.bfloat16)
    acc = [None] * G
    pend = [(0, tb), (1, tb1)]

    def wcb(k, g):
        return wc_ref[k, g].astype(jnp.bfloat16)

    def flush(pend, acc):
        for g in range(G):
            s = wcb(pend[0][0], g) * pend[0][1]
            for k, t in pend[1:]:
                s = s + wcb(k, g) * t
            s = s.astype(jnp.float32)
            acc[g] = s if acc[g] is None else acc[g] + s

    tb_m1 = tb1
    for k in range(2, K):
        t = dot(tb_m1, Lb2) - t_m2
        t_m2, t_m1 = t_m1, t
        tb_m1 = t.astype(jnp.bfloat16)
        pend.append((k, tb_m1))
        if len(pend) == CHUNK:
            flush(pend, acc)
            pend = []
    if pend:
        flush(pend, acc)

    wfcb = wfc_ref[...].astype(jnp.bfloat16)
    logits = bfc_ref[...]
    for g in range(G):
        h_g = jnp.maximum(acc[g] + bc_ref[g], 0.0)
        logits = logits + dot(h_g.astype(jnp.bfloat16), wfcb[g])

    m = jnp.max(logits, axis=1, keepdims=True)
    s = logits - m
    out_ref[...] = s - jnp.log(jnp.sum(jnp.exp(s), axis=1, keepdims=True))


def kernel(x, L, W_cheb, b_cheb, W_fc, b_fc):
    B, N, F_IN = x.shape
    K, _, G = W_cheb.shape
    D = W_fc.shape[1]
    xt = x.reshape(B, N)
    wc = W_cheb.reshape(K, G)
    wfc = W_fc.reshape(N, G, D).transpose(1, 0, 2)

    fn = pl.pallas_call(
        functools.partial(_fused_kernel, K=K, G=G),
        out_shape=jax.ShapeDtypeStruct((B, D), jnp.float32),
        in_specs=[
            pl.BlockSpec(memory_space=pltpu.VMEM),
            pl.BlockSpec(memory_space=pltpu.VMEM),
            pl.BlockSpec(memory_space=pltpu.SMEM),
            pl.BlockSpec(memory_space=pltpu.SMEM),
            pl.BlockSpec(memory_space=pltpu.VMEM),
            pl.BlockSpec(memory_space=pltpu.VMEM),
        ],
        out_specs=pl.BlockSpec(memory_space=pltpu.VMEM),
    )
    return fn(xt, L, wc, b_cheb, wfc, b_fc.reshape(1, D))


# speedup vs baseline: 1.1626x; 1.0109x over previous
import functools

import jax
import jax.numpy as jnp
from jax.experimental import pallas as pl
from jax.experimental.pallas import tpu as pltpu


def _fused_kernel(x_ref, L_ref, wc_ref, bc_ref, wfc_ref, bfc_ref, out_ref,
                  *, K, G):
    Lb2 = (L_ref[...] * 2.0).astype(jnp.bfloat16)
    dot = functools.partial(jnp.dot, preferred_element_type=jnp.float32)

    CHUNK = 8
    t_m2 = x_ref[...]
    tb = x_ref[...].astype(jnp.bfloat16)
    t_m1 = 0.5 * dot(tb, Lb2)
    tb1 = t_m1.astype(jnp.bfloat16)
    acc = [None] * G
    pend = [(0, tb), (1, tb1)]

    def wcb(k, g):
        return wc_ref[k, g].astype(jnp.bfloat16)

    def flush(pend, acc):
        for g in range(G):
            s = wcb(pend[0][0], g) * pend[0][1]
            for k, t in pend[1:]:
                s = s + wcb(k, g) * t
            s = s.astype(jnp.float32)
            acc[g] = s if acc[g] is None else acc[g] + s

    tb_m1 = tb1
    for k in range(2, K):
        t = dot(tb_m1, Lb2) - t_m2
        t_m2, t_m1 = t_m1, t
        tb_m1 = t.astype(jnp.bfloat16)
        pend.append((k, tb_m1))
        if len(pend) == CHUNK:
            flush(pend, acc)
            pend = []
    if pend:
        flush(pend, acc)

    wfcb = wfc_ref[...].astype(jnp.bfloat16)
    logits = bfc_ref[...]
    for g in range(G):
        h_g = jnp.maximum(acc[g] + bc_ref[g], 0.0)
        logits = logits + dot(h_g.astype(jnp.bfloat16), wfcb[g])

    m = jnp.max(logits, axis=1, keepdims=True)
    s = logits - m
    out_ref[...] = s - jnp.log(jnp.sum(jnp.exp(s), axis=1, keepdims=True))


def kernel(x, L, W_cheb, b_cheb, W_fc, b_fc):
    B, N, F_IN = x.shape
    K, _, G = W_cheb.shape
    D = W_fc.shape[1]
    xt = x.reshape(B, N)
    wc = W_cheb.reshape(K, G)
    wfc = W_fc.reshape(N, G, D).transpose(1, 0, 2)

    fn = pl.pallas_call(
        functools.partial(_fused_kernel, K=K, G=G),
        out_shape=jax.ShapeDtypeStruct((B, D), jnp.float32),
        in_specs=[
            pl.BlockSpec(memory_space=pltpu.VMEM),
            pl.BlockSpec(memory_space=pltpu.VMEM),
            pl.BlockSpec(memory_space=pltpu.SMEM),
            pl.BlockSpec(memory_space=pltpu.SMEM),
            pl.BlockSpec(memory_space=pltpu.VMEM),
            pl.BlockSpec(memory_space=pltpu.VMEM),
        ],
        out_specs=pl.BlockSpec(memory_space=pltpu.VMEM),
    )
    return fn(xt, L, wc, b_cheb, wfc, b_fc.reshape(1, D))


# speedup vs baseline: 1.1720x; 1.0081x over previous
import functools

import jax
import jax.numpy as jnp
from jax.experimental import pallas as pl
from jax.experimental.pallas import tpu as pltpu


def _fused_kernel(x_ref, L_ref, wc_ref, bc_ref, wfc_ref, bfc_ref, out_ref,
                  *, K, G):
    Lb2 = (L_ref[...] * 2.0).astype(jnp.bfloat16)
    dot = functools.partial(jnp.dot, preferred_element_type=jnp.float32)

    CHUNK = 12
    t_m2 = x_ref[...]
    tb = x_ref[...].astype(jnp.bfloat16)
    t_m1 = 0.5 * dot(tb, Lb2)
    tb1 = t_m1.astype(jnp.bfloat16)
    acc = [None] * G
    pend = [(0, tb), (1, tb1)]

    def wcb(k, g):
        return wc_ref[k, g].astype(jnp.bfloat16)

    def flush(pend, acc):
        for g in range(G):
            s = wcb(pend[0][0], g) * pend[0][1]
            for k, t in pend[1:]:
                s = s + wcb(k, g) * t
            s = s.astype(jnp.float32)
            acc[g] = s if acc[g] is None else acc[g] + s

    tb_m1 = tb1
    for k in range(2, K):
        t = dot(tb_m1, Lb2) - t_m2
        t_m2, t_m1 = t_m1, t
        tb_m1 = t.astype(jnp.bfloat16)
        pend.append((k, tb_m1))
        if len(pend) == CHUNK:
            flush(pend, acc)
            pend = []
    if pend:
        flush(pend, acc)

    wfcb = wfc_ref[...].astype(jnp.bfloat16)
    logits = bfc_ref[...]
    for g in range(G):
        h_g = jnp.maximum(acc[g] + bc_ref[g], 0.0)
        logits = logits + dot(h_g.astype(jnp.bfloat16), wfcb[g])

    m = jnp.max(logits, axis=1, keepdims=True)
    s = logits - m
    out_ref[...] = s - jnp.log(jnp.sum(jnp.exp(s), axis=1, keepdims=True))


def kernel(x, L, W_cheb, b_cheb, W_fc, b_fc):
    B, N, F_IN = x.shape
    K, _, G = W_cheb.shape
    D = W_fc.shape[1]
    xt = x.reshape(B, N)
    wc = W_cheb.reshape(K, G)
    wfc = W_fc.reshape(N, G, D).transpose(1, 0, 2)

    fn = pl.pallas_call(
        functools.partial(_fused_kernel, K=K, G=G),
        out_shape=jax.ShapeDtypeStruct((B, D), jnp.float32),
        in_specs=[
            pl.BlockSpec(memory_space=pltpu.VMEM),
            pl.BlockSpec(memory_space=pltpu.VMEM),
            pl.BlockSpec(memory_space=pltpu.SMEM),
            pl.BlockSpec(memory_space=pltpu.SMEM),
            pl.BlockSpec(memory_space=pltpu.VMEM),
            pl.BlockSpec(memory_space=pltpu.VMEM),
        ],
        out_specs=pl.BlockSpec(memory_space=pltpu.VMEM),
    )
    return fn(xt, L, wc, b_cheb, wfc, b_fc.reshape(1, D))
